# baseline (speedup 1.0000x reference)
import jax
import jax.numpy as jnp
from jax import lax
from jax.experimental import pallas as pl
from jax.experimental.pallas import tpu as pltpu
from jax.experimental.pallas import tpu_sc as plsc

NUM_CLUSTERS = 1000000
Z_N = 16
B, L = 4096, 200

NC, NS = 2, 16
NW = NC * NS
BW = B // NW
RG = 2
NG = BW // RG
SPANS = ((0, 128), (128, 72))


def _gather_body(x_hbm, mu_hbm, lv_hbm, mu_out, lv_out,
                 idx_v, mu_buf, lv_buf, sem):
    wid = lax.axis_index("s") * NC + lax.axis_index("c")
    row0 = wid * BW
    pltpu.sync_copy(x_hbm.at[pl.ds(row0 * L, BW * L)], idx_v)

    def group(g, carry):
        descs = []
        for jr in range(RG):
            r = g * RG + jr
            for c, w in SPANS:
                src_idx = idx_v.at[pl.ds(r * L + c, w)]
                dst = (jr, pl.ds(c, w))
                descs.append(
                    pltpu.async_copy(mu_hbm.at[src_idx], mu_buf.at[dst], sem))
                descs.append(
                    pltpu.async_copy(lv_hbm.at[src_idx], lv_buf.at[dst], sem))
        for d in descs:
            d.wait()
        out_sl = pl.ds(row0 + g * RG, RG)
        pltpu.sync_copy(mu_buf, mu_out.at[out_sl])
        pltpu.sync_copy(lv_buf, lv_out.at[out_sl])
        return carry

    lax.fori_loop(0, NG, group, 0)


def kernel(x, embed_mu, embed_logvar):
    mesh = plsc.VectorSubcoreMesh(core_axis_name="c", subcore_axis_name="s")
    kfn = pl.kernel(
        _gather_body,
        out_type=(
            jax.ShapeDtypeStruct((B, L, Z_N), jnp.float32),
            jax.ShapeDtypeStruct((B, L, Z_N), jnp.float32),
        ),
        mesh=mesh,
        compiler_params=pltpu.CompilerParams(use_tc_tiling_on_sc=False),
        scratch_types=[
            pltpu.VMEM((BW * L,), jnp.int32),
            pltpu.VMEM((RG, L, Z_N), jnp.float32),
            pltpu.VMEM((RG, L, Z_N), jnp.float32),
            pltpu.SemaphoreType.DMA,
        ],
    )
    mu, logvar = kfn(x.reshape(B * L), embed_mu, embed_logvar)
    return (mu, mu, logvar)


# speedup vs baseline: 1.2000x; 1.2000x over previous
import jax
import jax.numpy as jnp
from jax import lax
from jax.experimental import pallas as pl
from jax.experimental.pallas import tpu as pltpu
from jax.experimental.pallas import tpu_sc as plsc

NUM_CLUSTERS = 1000000
Z_N = 16
B, L = 4096, 200

NC, NS = 2, 16
NW = NC * NS
BW = B // NW
RG = 2
NG = BW // RG
GSZ = RG * L
SPANS = ((0, 128), (128, 72))


def _gather_body(x_hbm, mu_hbm, lv_hbm, mu_out, lv_out,
                 idx_v, mu_buf, lv_buf, sem):
    wid = lax.axis_index("s") * NC + lax.axis_index("c")
    row0 = wid * BW
    pltpu.sync_copy(x_hbm.at[pl.ds(row0 * L, BW * L)], idx_v)

    def group(g, carry):
        descs = []
        for jr in range(RG):
            r = g * RG + jr
            for c, w in SPANS:
                src_idx = idx_v.at[pl.ds(r * L + c, w)]
                dst = pl.ds(jr * L + c, w)
                descs.append(
                    pltpu.async_copy(mu_hbm.at[src_idx], mu_buf.at[dst], sem))
                descs.append(
                    pltpu.async_copy(lv_hbm.at[src_idx], lv_buf.at[dst], sem))
        for d in descs:
            d.wait()
        out_sl = pl.ds((row0 + g * RG) * L, GSZ)
        cs = pl.ds(0, Z_N)
        pltpu.sync_copy(mu_buf, mu_out.at[out_sl, cs])
        pltpu.sync_copy(lv_buf, lv_out.at[out_sl, cs])
        return carry

    lax.fori_loop(0, NG, group, 0)


def kernel(x, embed_mu, embed_logvar):
    mesh = plsc.VectorSubcoreMesh(core_axis_name="c", subcore_axis_name="s")
    kfn = pl.kernel(
        _gather_body,
        out_type=(
            jax.ShapeDtypeStruct((B * L, 128), jnp.float32),
            jax.ShapeDtypeStruct((B * L, 128), jnp.float32),
        ),
        mesh=mesh,
        compiler_params=pltpu.CompilerParams(use_tc_tiling_on_sc=False),
        scratch_types=[
            pltpu.VMEM((BW * L,), jnp.int32),
            pltpu.VMEM((GSZ, Z_N), jnp.float32),
            pltpu.VMEM((GSZ, Z_N), jnp.float32),
            pltpu.SemaphoreType.DMA,
        ],
    )
    mu_p, lv_p = kfn(x.reshape(B * L), embed_mu, embed_logvar)
    mu = mu_p[:, :Z_N].reshape(B, L, Z_N)
    logvar = lv_p[:, :Z_N].reshape(B, L, Z_N)
    return (mu, mu, logvar)


# speedup vs baseline: 1.2302x; 1.0252x over previous
import jax
import jax.numpy as jnp
from jax import lax
from jax.experimental import pallas as pl
from jax.experimental.pallas import tpu as pltpu
from jax.experimental.pallas import tpu_sc as plsc

NUM_CLUSTERS = 1000000
Z_N = 16
B, L = 4096, 200

NC, NS = 2, 16
NW = NC * NS
BW = B // NW
RG = 2
NG = BW // RG
GSZ = RG * L
SPANS = ((0, 128), (128, 72))


def _gather_body(x_hbm, mu_hbm, lv_hbm, mu_out, lv_out,
                 idx_v, mu_buf, lv_buf, sem):
    wid = lax.axis_index("s") * NC + lax.axis_index("c")
    row0 = wid * BW
    pltpu.sync_copy(x_hbm.at[pl.ds(row0 * L, BW * L)], idx_v)

    def fire(g, b):
        descs = []
        for jr in range(RG):
            r = g * RG + jr
            for c, w in SPANS:
                src_idx = idx_v.at[pl.ds(r * L + c, w)]
                dst = pl.ds(b * GSZ + jr * L + c, w)
                descs.append(
                    pltpu.async_copy(mu_hbm.at[src_idx], mu_buf.at[dst], sem))
                descs.append(
                    pltpu.async_copy(lv_hbm.at[src_idx], lv_buf.at[dst], sem))
        return descs

    def drain(g, b, descs):
        for d in descs:
            d.wait()
        out_sl = pl.ds((row0 + g * RG) * L, GSZ)
        cs = pl.ds(0, Z_N)
        bsl = pl.ds(b * GSZ, GSZ)
        pltpu.sync_copy(mu_buf.at[bsl], mu_out.at[out_sl, cs])
        pltpu.sync_copy(lv_buf.at[bsl], lv_out.at[out_sl, cs])

    def pair(gg, carry):
        g0 = gg * 2
        d0 = fire(g0, 0)
        d1 = fire(g0 + 1, 1)
        drain(g0, 0, d0)
        drain(g0 + 1, 1, d1)
        return carry

    lax.fori_loop(0, NG // 2, pair, 0)


def kernel(x, embed_mu, embed_logvar):
    mesh = plsc.VectorSubcoreMesh(core_axis_name="c", subcore_axis_name="s")
    kfn = pl.kernel(
        _gather_body,
        out_type=(
            jax.ShapeDtypeStruct((B * L, 128), jnp.float32),
            jax.ShapeDtypeStruct((B * L, 128), jnp.float32),
        ),
        mesh=mesh,
        compiler_params=pltpu.CompilerParams(use_tc_tiling_on_sc=False),
        scratch_types=[
            pltpu.VMEM((BW * L,), jnp.int32),
            pltpu.VMEM((2 * GSZ, Z_N), jnp.float32),
            pltpu.VMEM((2 * GSZ, Z_N), jnp.float32),
            pltpu.SemaphoreType.DMA,
        ],
    )
    mu_p, lv_p = kfn(x.reshape(B * L), embed_mu, embed_logvar)
    mu = mu_p[:, :Z_N].reshape(B, L, Z_N)
    logvar = lv_p[:, :Z_N].reshape(B, L, Z_N)
    return (mu, mu, logvar)


# speedup vs baseline: 1.2449x; 1.0119x over previous
import jax
import jax.numpy as jnp
from jax import lax
from jax.experimental import pallas as pl
from jax.experimental.pallas import tpu as pltpu
from jax.experimental.pallas import tpu_sc as plsc

NUM_CLUSTERS = 1000000
Z_N = 16
B, L = 4096, 200

NC, NS = 2, 16
NW = NC * NS
BW = B // NW
RG = 4
NG = BW // RG
GSZ = RG * L
SPANS = ((0, 128), (128, 72))


def _gather_body(x_hbm, mu_hbm, lv_hbm, mu_out, lv_out,
                 idx_v, mu_buf, lv_buf, sem):
    wid = lax.axis_index("s") * NC + lax.axis_index("c")
    row0 = wid * BW
    pltpu.sync_copy(x_hbm.at[pl.ds(row0 * L, BW * L)], idx_v)

    def fire(g, b):
        descs = []
        for jr in range(RG):
            r = g * RG + jr
            for c, w in SPANS:
                src_idx = idx_v.at[pl.ds(r * L + c, w)]
                dst = pl.ds(b * GSZ + jr * L + c, w)
                descs.append(
                    pltpu.async_copy(mu_hbm.at[src_idx], mu_buf.at[dst], sem))
                descs.append(
                    pltpu.async_copy(lv_hbm.at[src_idx], lv_buf.at[dst], sem))
        return descs

    def drain(g, b, descs):
        for d in descs:
            d.wait()
        out_sl = pl.ds((row0 + g * RG) * L, GSZ)
        cs = pl.ds(0, Z_N)
        bsl = pl.ds(b * GSZ, GSZ)
        pltpu.sync_copy(mu_buf.at[bsl], mu_out.at[out_sl, cs])
        pltpu.sync_copy(lv_buf.at[bsl], lv_out.at[out_sl, cs])

    def pair(gg, carry):
        g0 = gg * 2
        d0 = fire(g0, 0)
        d1 = fire(g0 + 1, 1)
        drain(g0, 0, d0)
        drain(g0 + 1, 1, d1)
        return carry

    lax.fori_loop(0, NG // 2, pair, 0)


def kernel(x, embed_mu, embed_logvar):
    mesh = plsc.VectorSubcoreMesh(core_axis_name="c", subcore_axis_name="s")
    kfn = pl.kernel(
        _gather_body,
        out_type=(
            jax.ShapeDtypeStruct((B * L, 128), jnp.float32),
            jax.ShapeDtypeStruct((B * L, 128), jnp.float32),
        ),
        mesh=mesh,
        compiler_params=pltpu.CompilerParams(use_tc_tiling_on_sc=False),
        scratch_types=[
            pltpu.VMEM((BW * L,), jnp.int32),
            pltpu.VMEM((2 * GSZ, Z_N), jnp.float32),
            pltpu.VMEM((2 * GSZ, Z_N), jnp.float32),
            pltpu.SemaphoreType.DMA,
        ],
    )
    mu_p, lv_p = kfn(x.reshape(B * L), embed_mu, embed_logvar)
    mu = mu_p.reshape(B, L, 128)[:, :, :Z_N]
    logvar = lv_p.reshape(B, L, 128)[:, :, :Z_N]
    return (mu, mu, logvar)


# speedup vs baseline: 1.2727x; 1.0224x over previous
import jax
import jax.numpy as jnp
from jax import lax
from jax.experimental import pallas as pl
from jax.experimental.pallas import tpu as pltpu
from jax.experimental.pallas import tpu_sc as plsc

NUM_CLUSTERS = 1000000
Z_N = 16
B, L = 4096, 200

NC, NS = 2, 16
NW = NC * NS
BW = B // NW
RG = 4
NG = BW // RG
GSZ = RG * L
SPANS = ((0, 128), (128, 72))


def _gather_body(x_hbm, tab_hbm, out, idx_v, buf, sem):
    wid = lax.axis_index("s") * NC + lax.axis_index("c")
    row0 = wid * BW
    pltpu.sync_copy(x_hbm.at[pl.ds(row0 * L, BW * L)], idx_v)

    def fire(g, b):
        descs = []
        for jr in range(RG):
            r = g * RG + jr
            for c, w in SPANS:
                src_idx = idx_v.at[pl.ds(r * L + c, w)]
                dst = pl.ds(b * GSZ + jr * L + c, w)
                descs.append(
                    pltpu.async_copy(tab_hbm.at[src_idx], buf.at[dst], sem))
        return descs

    def drain(g, b, descs):
        for d in descs:
            d.wait()
        out_sl = pl.ds((row0 + g * RG) * L, GSZ)
        pltpu.sync_copy(buf.at[pl.ds(b * GSZ, GSZ)],
                        out.at[out_sl, pl.ds(0, Z_N)])

    def pair(gg, carry):
        g0 = gg * 2
        d0 = fire(g0, 0)
        d1 = fire(g0 + 1, 1)
        drain(g0, 0, d0)
        drain(g0 + 1, 1, d1)
        return carry

    lax.fori_loop(0, NG // 2, pair, 0)


def _make_gather():
    mesh = plsc.VectorSubcoreMesh(core_axis_name="c", subcore_axis_name="s")
    return pl.kernel(
        _gather_body,
        out_type=jax.ShapeDtypeStruct((B * L, 128), jnp.float32),
        mesh=mesh,
        compiler_params=pltpu.CompilerParams(use_tc_tiling_on_sc=False),
        scratch_types=[
            pltpu.VMEM((BW * L,), jnp.int32),
            pltpu.VMEM((2 * GSZ, Z_N), jnp.float32),
            pltpu.SemaphoreType.DMA,
        ],
    )


def kernel(x, embed_mu, embed_logvar):
    x1 = x.reshape(B * L)
    mu_p = _make_gather()(x1, embed_mu)
    lv_p = _make_gather()(x1, embed_logvar)
    mu = mu_p.reshape(B, L, 128)[:, :, :Z_N]
    logvar = lv_p.reshape(B, L, 128)[:, :, :Z_N]
    return (mu, mu, logvar)
